# baseline (speedup 1.0000x reference)
import functools

import jax
import jax.numpy as jnp
from jax import lax
from jax.experimental import pallas as pl
from jax.experimental.pallas import tpu as pltpu
from jax.experimental.pallas import tpu_sc as plsc

N = 10000
E = 320000
D = 128
DH = D // 2

NC = 2
NS = 16
NW = NC * NS
CH = 125
KH = E // (NW * CH)
C = 125
KA = E // (NS * C)
NB = 4
NP = 10240
RPS = NP // NS

_mesh = plsc.VectorSubcoreMesh(core_axis_name="c", subcore_axis_name="s")


@functools.partial(
    pl.kernel,
    out_type=jax.ShapeDtypeStruct((NC, NP, 16), jnp.float32),
    mesh=_mesh,
    scratch_types=[
        pltpu.VMEM((KH, CH), jnp.int32),
        pltpu.VMEM((CH, 16), jnp.float32),
        pltpu.VMEM_SHARED((NP, 16), jnp.float32),
    ],
    compiler_params=pltpu.CompilerParams(use_tc_tiling_on_sc=False),
)
def _sc_hist(dst_hbm, ones_hbm, zeros_hbm, hist_hbm, idx_v, ones_v, hist_sh):
    c = lax.axis_index("c")
    s = lax.axis_index("s")
    pltpu.sync_copy(zeros_hbm.at[pl.ds(s * RPS, RPS)],
                    hist_sh.at[pl.ds(s * RPS, RPS)])
    pltpu.sync_copy(dst_hbm.at[c, s], idx_v)
    pltpu.sync_copy(ones_hbm, ones_v)
    plsc.subcore_barrier()

    @pl.loop(0, KH)
    def _(k):
        pltpu.sync_copy(ones_v, hist_sh.at[idx_v.at[k]], add=True)

    plsc.subcore_barrier()
    pltpu.sync_copy(hist_sh.at[pl.ds(s * RPS, RPS)],
                    hist_hbm.at[c, pl.ds(s * RPS, RPS)])


@functools.partial(
    pl.kernel,
    out_type=jax.ShapeDtypeStruct((NC, NP, DH), jnp.float32),
    mesh=_mesh,
    scratch_types=[
        pltpu.VMEM((KA, C), jnp.int32),
        pltpu.VMEM((KA, C), jnp.int32),
        pltpu.VMEM((C, DH), jnp.float32),
        pltpu.VMEM((C, DH), jnp.float32),
        pltpu.VMEM((C, DH), jnp.float32),
        pltpu.VMEM((C, DH), jnp.float32),
        pltpu.VMEM_SHARED((NP, DH), jnp.float32),
        pltpu.SemaphoreType.DMA,
        pltpu.SemaphoreType.DMA,
        pltpu.SemaphoreType.DMA,
        pltpu.SemaphoreType.DMA,
        pltpu.SemaphoreType.DMA,
        pltpu.SemaphoreType.DMA,
        pltpu.SemaphoreType.DMA,
        pltpu.SemaphoreType.DMA,
    ],
    compiler_params=pltpu.CompilerParams(use_tc_tiling_on_sc=False),
)
def _sc_aggregate(g_hbm, ei_hbm, acc_hbm,
                  src_v, dst_v, r0, r1, r2, r3, acc_sh,
                  g0, g1, g2, g3, s0, s1, s2, s3):
    rows = (r0, r1, r2, r3)
    gs = (g0, g1, g2, g3)
    ss = (s0, s1, s2, s3)
    c = lax.axis_index("c")
    s = lax.axis_index("s")
    pltpu.sync_copy(ei_hbm.at[s], src_v)
    pltpu.sync_copy(ei_hbm.at[NS + s], dst_v)

    pltpu.sync_copy(g_hbm.at[c, pl.ds(s * RPS, RPS)],
                    acc_sh.at[pl.ds(s * RPS, RPS)])
    plsc.subcore_barrier()

    gc = g_hbm.at[c]
    for j in range(NB):
        pltpu.async_copy(gc.at[src_v.at[j]], rows[j], gs[j])

    @pl.loop(0, KA, step=NB)
    def _(k):
        for j in range(NB):
            pltpu.make_async_copy(gc.at[src_v.at[k + j]], rows[j],
                                  gs[j]).wait()
            pltpu.async_copy(rows[j], acc_sh.at[dst_v.at[k + j]], ss[j],
                             add=True)
        for j in range(NB):
            @pl.when(k + j + NB < KA)
            def _(j=j):
                pltpu.make_async_copy(rows[j], acc_sh.at[dst_v.at[k + j]],
                                      ss[j]).wait()
                pltpu.async_copy(gc.at[src_v.at[k + j + NB]], rows[j], gs[j])

    for j in range(NB):
        pltpu.make_async_copy(rows[j], acc_sh.at[dst_v.at[KA - NB + j]],
                              ss[j]).wait()

    plsc.subcore_barrier()
    pltpu.sync_copy(acc_sh.at[pl.ds(s * RPS, RPS)],
                    acc_hbm.at[c, pl.ds(s * RPS, RPS)])


def _tc_prep_body(emb_ref, w_ref, hist_ref, g_ref):
    hw = jnp.dot(emb_ref[...], w_ref[...], preferred_element_type=jnp.float32)
    deg = 1.0 + hist_ref[0, :N, 0] + hist_ref[1, :N, 0]
    dis = lax.rsqrt(deg)
    gd = jnp.pad(hw * dis[:, None], ((0, NP - N), (0, 0)))
    g_ref[...] = jnp.stack([gd[:, :DH], gd[:, DH:]], axis=0)


def _tc_final_body(acc_ref, hist_ref, b_ref, o_ref):
    deg = 1.0 + hist_ref[0, :N, 0] + hist_ref[1, :N, 0]
    dis = lax.rsqrt(deg)
    agg = jnp.concatenate([acc_ref[0, :N], acc_ref[1, :N]], axis=1)
    o_ref[...] = agg * dis[:, None] + b_ref[...]


def kernel(x, edge_index, emb_weight, W, b):
    del x
    ei = edge_index.reshape(2 * NS, KA, C)
    dst_h = ei[NS:].reshape(NC, NS, KH, CH)

    ones16 = jnp.ones((CH, 16), jnp.float32)
    zeros16 = jnp.zeros((NP, 16), jnp.float32)

    hist = _sc_hist(dst_h, ones16, zeros16)

    g = pl.pallas_call(
        _tc_prep_body,
        out_shape=jax.ShapeDtypeStruct((NC, NP, DH), jnp.float32),
    )(emb_weight, W, hist)

    acc = _sc_aggregate(g, ei)

    out = pl.pallas_call(
        _tc_final_body,
        out_shape=jax.ShapeDtypeStruct((N, D), jnp.float32),
    )(acc, hist, b.reshape(1, D))
    return out


# speedup vs baseline: 1.0317x; 1.0317x over previous
import functools

import jax
import jax.numpy as jnp
from jax import lax
from jax.experimental import pallas as pl
from jax.experimental.pallas import tpu as pltpu
from jax.experimental.pallas import tpu_sc as plsc

N = 10000
E = 320000
D = 128
DH = D // 2

NC = 2
NS = 16
NW = NC * NS
CH = 125
KH = E // (NW * CH)
C = 125
KA = E // (NS * C)
NB = 4
NP = 10240
RPS = NP // NS

_mesh = plsc.VectorSubcoreMesh(core_axis_name="c", subcore_axis_name="s")


@functools.partial(
    pl.kernel,
    out_type=jax.ShapeDtypeStruct((NC, NP, 16), jnp.float32),
    mesh=_mesh,
    scratch_types=[
        pltpu.VMEM((KH, CH), jnp.int32),
        pltpu.VMEM((CH, 16), jnp.float32),
        pltpu.VMEM_SHARED((NP, 16), jnp.float32),
    ],
    compiler_params=pltpu.CompilerParams(use_tc_tiling_on_sc=False),
)
def _sc_hist(dst_hbm, ones_hbm, zeros_hbm, hist_hbm, idx_v, ones_v, hist_sh):
    c = lax.axis_index("c")
    s = lax.axis_index("s")
    pltpu.sync_copy(zeros_hbm.at[pl.ds(s * RPS, RPS)],
                    hist_sh.at[pl.ds(s * RPS, RPS)])
    pltpu.sync_copy(dst_hbm.at[c, s], idx_v)
    pltpu.sync_copy(ones_hbm, ones_v)
    plsc.subcore_barrier()

    @pl.loop(0, KH)
    def _(k):
        pltpu.sync_copy(ones_v, hist_sh.at[idx_v.at[k]], add=True)

    plsc.subcore_barrier()
    pltpu.sync_copy(hist_sh.at[pl.ds(s * RPS, RPS)],
                    hist_hbm.at[c, pl.ds(s * RPS, RPS)])


@functools.partial(
    pl.kernel,
    out_type=jax.ShapeDtypeStruct((NC, NP, DH), jnp.float32),
    mesh=_mesh,
    scratch_types=[
        pltpu.VMEM((KA, C), jnp.int32),
        pltpu.VMEM((KA, C), jnp.int32),
        pltpu.VMEM((C, DH), jnp.float32),
        pltpu.VMEM((C, DH), jnp.float32),
        pltpu.VMEM((C, DH), jnp.float32),
        pltpu.VMEM((C, DH), jnp.float32),
        pltpu.VMEM_SHARED((NP, DH), jnp.float32),
        pltpu.SemaphoreType.DMA,
        pltpu.SemaphoreType.DMA,
        pltpu.SemaphoreType.DMA,
        pltpu.SemaphoreType.DMA,
        pltpu.SemaphoreType.DMA,
        pltpu.SemaphoreType.DMA,
        pltpu.SemaphoreType.DMA,
        pltpu.SemaphoreType.DMA,
    ],
    compiler_params=pltpu.CompilerParams(use_tc_tiling_on_sc=False),
)
def _sc_aggregate(g_hbm, zeros_hbm, ei_hbm, acc_hbm,
                  src_v, dst_v, r0, r1, r2, r3, acc_sh,
                  g0, g1, g2, g3, s0, s1, s2, s3):
    rows = (r0, r1, r2, r3)
    gs = (g0, g1, g2, g3)
    ss = (s0, s1, s2, s3)
    c = lax.axis_index("c")
    s = lax.axis_index("s")
    pltpu.sync_copy(ei_hbm.at[c * NS + s], src_v)
    pltpu.sync_copy(ei_hbm.at[2 * NS + s], dst_v)

    pltpu.sync_copy(zeros_hbm.at[pl.ds(s * RPS, RPS)],
                    acc_sh.at[pl.ds(s * RPS, RPS)])
    plsc.subcore_barrier()

    for j in range(NB):
        pltpu.async_copy(g_hbm.at[src_v.at[j]], rows[j], gs[j])

    @pl.loop(0, KA, step=NB)
    def _(k):
        for j in range(NB):
            pltpu.make_async_copy(g_hbm.at[src_v.at[k + j]], rows[j],
                                  gs[j]).wait()
            pltpu.async_copy(rows[j], acc_sh.at[dst_v.at[k + j]], ss[j],
                             add=True)
        for j in range(NB):
            @pl.when(k + j + NB < KA)
            def _(j=j):
                pltpu.make_async_copy(rows[j], acc_sh.at[dst_v.at[k + j]],
                                      ss[j]).wait()
                pltpu.async_copy(g_hbm.at[src_v.at[k + j + NB]], rows[j], gs[j])

    for j in range(NB):
        pltpu.make_async_copy(rows[j], acc_sh.at[dst_v.at[KA - NB + j]],
                              ss[j]).wait()

    plsc.subcore_barrier()
    pltpu.sync_copy(acc_sh.at[pl.ds(s * RPS, RPS)],
                    acc_hbm.at[c, pl.ds(s * RPS, RPS)])


def _dis_from_hist128(hist_ref):
    h = hist_ref[...].reshape(NC, NP // 8, 8, 16)[:, :, :, 0]
    deg = 1.0 + (h[0] + h[1]).reshape(NP)[:N]
    return lax.rsqrt(deg)


def _tc_prep_body(emb_ref, w_ref, hist_ref, g_ref, disb_ref):
    hw = jnp.dot(emb_ref[...], w_ref[...], preferred_element_type=jnp.float32)
    dis = _dis_from_hist128(hist_ref)
    g_ref[...] = jnp.pad(hw * dis[:, None], ((0, NP - N), (0, 0)))
    disb_ref[...] = jnp.pad(jnp.broadcast_to(dis[:, None], (N, D)),
                            ((0, NP - N), (0, 0)))


RB = 2000


def _tc_final_body(acc_ref, g_ref, disb_ref, b_ref, o_ref):
    av = acc_ref[...]
    a0 = jnp.stack([av[0, :, :DH], av[0, :, DH:]], axis=1).reshape(RB, DH)
    a1 = jnp.stack([av[1, :, :DH], av[1, :, DH:]], axis=1).reshape(RB, DH)
    agg = jnp.concatenate([a0, a1], axis=1) + g_ref[...]
    o_ref[...] = agg * disb_ref[...] + b_ref[...]


def kernel(x, edge_index, emb_weight, W, b):
    del x
    s2 = edge_index[0] * 2
    ei = jnp.stack([s2, s2 + 1, edge_index[1]]).reshape(3 * NS, KA, C)
    dst_h = ei[2 * NS:].reshape(NC, NS, KH, CH)

    ones16 = jnp.ones((CH, 16), jnp.float32)
    zeros16 = jnp.zeros((NP, 16), jnp.float32)
    zerosD = jnp.zeros((NP, DH), jnp.float32)

    hist = _sc_hist(dst_h, ones16, zeros16)
    hist_r = hist.reshape(NC, NP * 16 // D, D)

    g, disb = pl.pallas_call(
        _tc_prep_body,
        out_shape=(jax.ShapeDtypeStruct((NP, D), jnp.float32),
                   jax.ShapeDtypeStruct((NP, D), jnp.float32)),
    )(emb_weight, W, hist_r)
    g_il = g.reshape(2 * NP, DH)

    acc = _sc_aggregate(g_il, zerosD, ei)
    acc_r = acc.reshape(NC, NP // 2, D)

    out = pl.pallas_call(
        _tc_final_body,
        grid=(N // RB,),
        in_specs=[
            pl.BlockSpec((NC, RB // 2, D), lambda i: (0, i, 0)),
            pl.BlockSpec((RB, D), lambda i: (i, 0)),
            pl.BlockSpec((RB, D), lambda i: (i, 0)),
            pl.BlockSpec((1, D), lambda i: (0, 0)),
        ],
        out_specs=pl.BlockSpec((RB, D), lambda i: (i, 0)),
        out_shape=jax.ShapeDtypeStruct((N, D), jnp.float32),
    )(acc_r, g, disb, b.reshape(1, D))
    return out
